# baseline (speedup 1.0000x reference)
import functools

import jax
import jax.numpy as jnp
from jax import lax
from jax.experimental import pallas as pl
from jax.experimental.pallas import tpu as pltpu
from jax.experimental.pallas import tpu_sc as plsc

B = 16384
D = 64
NC = 2
NS = 16
NW = NC * NS
BPW = B // NW
CHUNK = 128
NCH = BPW // CHUNK
NGRP = BPW // 16


def _bprmf_body(users_r, pos_r, neg_r, ut_r, it_r,
                pos_s_out, neg_s_out, u_out, p_out, n_out,
                idx_u, idx_p, idx_n, u_rows, p_rows, n_rows,
                sc_p, sc_n, sem):
    c = lax.axis_index("c")
    s = lax.axis_index("s")
    wid = s * NC + c
    base = wid * BPW

    pltpu.sync_copy(users_r.at[pl.ds(wid * NCH, NCH)], idx_u)
    pltpu.sync_copy(pos_r.at[pl.ds(wid * NCH, NCH)], idx_p)
    pltpu.sync_copy(neg_r.at[pl.ds(wid * NCH, NCH)], idx_n)

    cps = []
    for j in range(NCH):
        dst = pl.ds(j * CHUNK, CHUNK)
        cps.append(pltpu.async_copy(ut_r.at[idx_u.at[j]], u_rows.at[dst], sem))
        cps.append(pltpu.async_copy(it_r.at[idx_p.at[j]], p_rows.at[dst], sem))
        cps.append(pltpu.async_copy(it_r.at[idx_n.at[j]], n_rows.at[dst], sem))
    for cp in cps:
        cp.wait()

    iota16 = lax.iota(jnp.int32, 16)
    zero16 = jnp.zeros((16,), jnp.float32)
    for g in range(NGRP):
        rows_i = g * 16 + iota16

        def dbody(d, carry, rows_i=rows_i):
            ap, an = carry
            cols = jnp.broadcast_to(d, (16,)).astype(jnp.int32)
            uc = plsc.load_gather(u_rows, [rows_i, cols])
            pc = plsc.load_gather(p_rows, [rows_i, cols])
            nc = plsc.load_gather(n_rows, [rows_i, cols])
            return (ap + uc * pc, an + uc * nc)

        ap, an = lax.fori_loop(0, D, dbody, (zero16, zero16))
        sc_p[pl.ds(g * 16, 16)] = ap
        sc_n[pl.ds(g * 16, 16)] = an

    out_sl = pl.ds(base, BPW)
    pltpu.sync_copy(u_rows, u_out.at[out_sl])
    pltpu.sync_copy(p_rows, p_out.at[out_sl])
    pltpu.sync_copy(n_rows, n_out.at[out_sl])
    pltpu.sync_copy(sc_p, pos_s_out.at[out_sl])
    pltpu.sync_copy(sc_n, neg_s_out.at[out_sl])


@jax.jit
def _bprmf(users2, pos2, neg2, user_table, item_table):
    mesh = plsc.VectorSubcoreMesh(core_axis_name="c", subcore_axis_name="s",
                                  num_cores=NC, num_subcores=NS)
    f32 = jnp.float32
    out_type = (
        jax.ShapeDtypeStruct((B,), f32),
        jax.ShapeDtypeStruct((B,), f32),
        jax.ShapeDtypeStruct((B, D), f32),
        jax.ShapeDtypeStruct((B, D), f32),
        jax.ShapeDtypeStruct((B, D), f32),
    )
    scratch = [
        pltpu.VMEM((NCH, CHUNK), jnp.int32),
        pltpu.VMEM((NCH, CHUNK), jnp.int32),
        pltpu.VMEM((NCH, CHUNK), jnp.int32),
        pltpu.VMEM((BPW, D), f32),
        pltpu.VMEM((BPW, D), f32),
        pltpu.VMEM((BPW, D), f32),
        pltpu.VMEM((BPW,), f32),
        pltpu.VMEM((BPW,), f32),
        pltpu.SemaphoreType.DMA,
    ]
    run = pl.kernel(_bprmf_body, out_type=out_type, mesh=mesh,
                    scratch_types=scratch,
                    compiler_params=pltpu.CompilerParams(
                        needs_layout_passes=False,
                        use_tc_tiling_on_sc=False))
    return run(users2, pos2, neg2, user_table, item_table)


def kernel(users, pos_items, neg_items, user_table, item_table):
    users2 = users.astype(jnp.int32).reshape(NW * NCH, CHUNK)
    pos2 = pos_items.astype(jnp.int32).reshape(NW * NCH, CHUNK)
    neg2 = neg_items.astype(jnp.int32).reshape(NW * NCH, CHUNK)
    return _bprmf(users2, pos2, neg2, user_table, item_table)
